# baseline (speedup 1.0000x reference)
import functools

import jax
import jax.numpy as jnp
from jax import lax
from jax.experimental import pallas as pl
from jax.experimental.pallas import tpu as pltpu
from jax.experimental.pallas import tpu_sc as plsc

N0 = 3072; N1 = 30720; K = 10; U = 33792; NN = 100000; B = 1024
TD = 32; ED = 16; D = 128; H = 2; DH = D // H
DQ = D + TD; DKV = D + ED + TD

NW = 32
C1 = N1 // NW
C0 = N0 // NW
UW = U // NW

f32 = jnp.float32
i32 = jnp.int32

@functools.lru_cache(maxsize=1)
def _mesh():
    return plsc.VectorSubcoreMesh(core_axis_name="c", subcore_axis_name="s")


def _wid():
    return lax.axis_index("s") * 2 + lax.axis_index("c")


def _sc_keys_body(nids1_h, nids0_h, n2i_h, key1_h, key0_h,
                  nbuf, ibuf, n0buf, i0buf, sem):
    w = _wid()
    b1 = w * C1
    pltpu.sync_copy(nids1_h.at[pl.ds(b1, C1)], nbuf)
    pltpu.async_copy(n2i_h.at[nbuf], ibuf, sem).wait()
    pltpu.sync_copy(ibuf, key1_h.at[pl.ds(b1, C1)])

    b0 = w * C0
    pltpu.sync_copy(nids0_h.at[pl.ds(b0, C0)], n0buf)
    pltpu.async_copy(n2i_h.at[n0buf], i0buf, sem).wait()
    pltpu.sync_copy(i0buf, key0_h.at[pl.ds(b0, C0)])


@functools.lru_cache(maxsize=1)
def _sc_keys():
    return pl.kernel(
        _sc_keys_body,
        out_type=[jax.ShapeDtypeStruct((N1,), i32),
                  jax.ShapeDtypeStruct((N0,), i32)],
        mesh=_mesh(),
        compiler_params=pltpu.CompilerParams(needs_layout_passes=False),
        scratch_types=[pltpu.VMEM((C1,), i32), pltpu.VMEM((C1,), i32),
                       pltpu.VMEM((C0,), i32), pltpu.VMEM((C0,), i32),
                       pltpu.SemaphoreType.DMA],
    )


def _sc_winner_body(key1_h, key0_h, w1_h, w0_h, kbuf, k0buf, w1loc, w0loc):
    w = _wid()
    lo = w * UW
    pltpu.sync_copy(key1_h, kbuf)
    pltpu.sync_copy(key0_h, k0buf)
    neg1 = jnp.full((16,), -1, i32)

    def init(c, _):
        w1loc[pl.ds(c * 16, 16)] = neg1
        w0loc[pl.ds(c * 16, 16)] = neg1
        return 0

    lax.fori_loop(0, UW // 16, init, 0)

    iot = lax.iota(i32, 16)

    def scan(buf, wloc, nch):
        def body(c, _):
            d = buf[pl.ds(c * 16, 16)]
            j = c * 16 + iot
            m = (d >= lo) & (d < lo + UW)
            dl = jnp.clip(d - lo, 0, UW - 1)
            plsc.store_scatter(wloc, [dl], j, mask=m)
            for _ in range(4):
                cur = plsc.load_gather(wloc, [dl], mask=m)
                plsc.store_scatter(wloc, [dl], j, mask=m & (cur < j))
            return 0

        lax.fori_loop(0, nch, body, 0)

    scan(kbuf, w1loc, N1 // 16)
    scan(k0buf, w0loc, N0 // 16)
    pltpu.sync_copy(w1loc, w1_h.at[pl.ds(lo, UW)])
    pltpu.sync_copy(w0loc, w0_h.at[pl.ds(lo, UW)])


@functools.lru_cache(maxsize=1)
def _sc_winner():
    return pl.kernel(
        _sc_winner_body,
        out_type=[jax.ShapeDtypeStruct((U,), i32),
                  jax.ShapeDtypeStruct((U,), i32)],
        mesh=_mesh(),
        compiler_params=pltpu.CompilerParams(needs_layout_passes=False),
        scratch_types=[pltpu.VMEM((N1,), i32), pltpu.VMEM((N0,), i32),
                       pltpu.VMEM((UW,), i32), pltpu.VMEM((UW,), i32)],
    )


_RG = 240


def _sc_compose_body(w1_h, w0_h, nbrf_h, sel3_h, out1_h,
                     g0_h, rows_h, fs0_h, fs1_h,
                     nfbuf, g0buf, gibuf, selbuf, f0buf, f1buf, rowbuf, sem):
    w = _wid()
    b1 = w * C1
    pltpu.sync_copy(nbrf_h.at[pl.ds(b1, C1)], nfbuf)
    pltpu.async_copy(w1_h.at[nfbuf], g0buf, sem).wait()
    pltpu.sync_copy(g0buf, g0_h.at[pl.ds(b1, C1)])

    def clampb(c, _):
        gibuf[pl.ds(c * 16, 16)] = jnp.maximum(g0buf[pl.ds(c * 16, 16)], 0)
        return 0

    lax.fori_loop(0, C1 // 16, clampb, 0)

    for s in range(C1 // _RG):
        idx = gibuf.at[pl.ds(s * _RG, _RG)]
        pltpu.async_copy(out1_h.at[idx], rowbuf, sem).wait()
        pltpu.sync_copy(rowbuf, rows_h.at[pl.ds(b1 + s * _RG, _RG)])

    b0 = w * C0
    pltpu.sync_copy(sel3_h.at[pl.ds(b0, C0)], selbuf)
    pltpu.async_copy(w0_h.at[selbuf], f0buf, sem).wait()
    pltpu.async_copy(w1_h.at[selbuf], f1buf, sem).wait()
    pltpu.sync_copy(f0buf, fs0_h.at[pl.ds(b0, C0)])
    pltpu.sync_copy(f1buf, fs1_h.at[pl.ds(b0, C0)])


@functools.lru_cache(maxsize=1)
def _sc_compose():
    return pl.kernel(
        _sc_compose_body,
        out_type=[jax.ShapeDtypeStruct((N1,), i32),
                  jax.ShapeDtypeStruct((N1, D), f32),
                  jax.ShapeDtypeStruct((N0,), i32),
                  jax.ShapeDtypeStruct((N0,), i32)],
        mesh=_mesh(),
        compiler_params=pltpu.CompilerParams(needs_layout_passes=False),
        scratch_types=[pltpu.VMEM((C1,), i32), pltpu.VMEM((C1,), i32),
                       pltpu.VMEM((C1,), i32), pltpu.VMEM((C0,), i32),
                       pltpu.VMEM((C0,), i32), pltpu.VMEM((C0,), i32),
                       pltpu.VMEM((_RG, D), f32),
                       pltpu.SemaphoreType.DMA],
    )


def _sc_fgather_body(fs0_h, fs1_h, out0_h, out1_h, r0_h, r1_h,
                     f0buf, f1buf, c0buf, c1buf, rb0, rb1, sem):
    w = _wid()
    b = w * C0
    pltpu.sync_copy(fs0_h.at[pl.ds(b, C0)], f0buf)
    pltpu.sync_copy(fs1_h.at[pl.ds(b, C0)], f1buf)

    def clampb(c, _):
        c0buf[pl.ds(c * 16, 16)] = jnp.maximum(f0buf[pl.ds(c * 16, 16)], 0)
        c1buf[pl.ds(c * 16, 16)] = jnp.maximum(f1buf[pl.ds(c * 16, 16)], 0)
        return 0

    lax.fori_loop(0, C0 // 16, clampb, 0)
    pltpu.async_copy(out0_h.at[c0buf], rb0, sem).wait()
    pltpu.sync_copy(rb0, r0_h.at[pl.ds(b, C0)])
    pltpu.async_copy(out1_h.at[c1buf], rb1, sem).wait()
    pltpu.sync_copy(rb1, r1_h.at[pl.ds(b, C0)])


@functools.lru_cache(maxsize=1)
def _sc_fgather():
    return pl.kernel(
        _sc_fgather_body,
        out_type=[jax.ShapeDtypeStruct((N0, D), f32),
                  jax.ShapeDtypeStruct((N0, D), f32)],
        mesh=_mesh(),
        compiler_params=pltpu.CompilerParams(needs_layout_passes=False),
        scratch_types=[pltpu.VMEM((C0,), i32), pltpu.VMEM((C0,), i32),
                       pltpu.VMEM((C0,), i32), pltpu.VMEM((C0,), i32),
                       pltpu.VMEM((C0, D), f32), pltpu.VMEM((C0, D), f32),
                       pltpu.SemaphoreType.DMA],
    )


NB1 = 1024


def _hop1_body(t_ref, nt_ref, ef_ref, m_ref,
               wst_ref, wse_ref, wt_ref, bt_ref,
               r32_ref, r16_ref, st_ref, se_ref,
               wm_ref, b1_ref, w2_ref, b2_ref,
               out_ref, tf_scr):
    t = t_ref[:]
    for k in range(K):
        dtk = t - nt_ref[:, k:k + 1]
        tf_scr[:, k * TD:(k + 1) * TD] = jnp.cos(dtk * wt_ref[:] + bt_ref[:])
    tf = tf_scr[:]
    ef = ef_ref[:]
    scores = (jnp.dot(tf, wst_ref[:], preferred_element_type=f32)
              + jnp.dot(ef, wse_ref[:], preferred_element_type=f32))
    valid = m_ref[:] != 0
    ctxs = []
    for h in range(H):
        s_h = jnp.where(valid, scores[:, h * K:(h + 1) * K], -1e9)
        mx = jnp.max(s_h, axis=1, keepdims=True)
        p = jnp.exp(s_h - mx)
        a_h = p / jnp.sum(p, axis=1, keepdims=True)
        rep32 = jnp.dot(a_h, r32_ref[:], preferred_element_type=f32)
        rep16 = jnp.dot(a_h, r16_ref[:], preferred_element_type=f32)
        ct = jnp.dot(tf * rep32, st_ref[:], preferred_element_type=f32)
        ce = jnp.dot(ef * rep16, se_ref[:], preferred_element_type=f32)
        ctxs += [ce, ct]
    ctx2 = jnp.concatenate(ctxs, axis=1)
    y = jnp.maximum(jnp.dot(ctx2, wm_ref[:], preferred_element_type=f32)
                    + b1_ref[:], 0.0)
    out_ref[:] = jnp.dot(y, w2_ref[:], preferred_element_type=f32) + b2_ref[:]


def _hop1_call(t2d, nt, ef, msk, wst, wse, wt, bt, r32, r16, st, se,
               wm, b1, w2, b2):
    nblk = N1 // NB1
    full = lambda i: (0, 0)
    blk = lambda i: (i, 0)
    return pl.pallas_call(
        _hop1_body,
        grid=(nblk,),
        in_specs=[
            pl.BlockSpec((NB1, 1), blk), pl.BlockSpec((NB1, K), blk),
            pl.BlockSpec((NB1, K * ED), blk), pl.BlockSpec((NB1, K), blk),
            pl.BlockSpec((K * TD, H * K), full), pl.BlockSpec((K * ED, H * K), full),
            pl.BlockSpec((1, TD), full), pl.BlockSpec((1, TD), full),
            pl.BlockSpec((K, K * TD), full), pl.BlockSpec((K, K * ED), full),
            pl.BlockSpec((K * TD, TD), full), pl.BlockSpec((K * ED, ED), full),
            pl.BlockSpec((2 * (ED + TD), D), full), pl.BlockSpec((1, D), full),
            pl.BlockSpec((D, D), full), pl.BlockSpec((1, D), full),
        ],
        out_specs=pl.BlockSpec((NB1, D), blk),
        out_shape=jax.ShapeDtypeStruct((N1, D), f32),
        scratch_shapes=[pltpu.VMEM((NB1, K * TD), f32)],
    )(t2d, nt, ef, msk, wst, wse, wt, bt, r32, r16, st, se, wm, b1, w2, b2)


NB0 = 512


def _hop0_body(t_ref, nt_ref, ef_ref, m_ref, g_ref, zr_ref,
               wsz_ref, wst_ref, wse_ref, wt_ref, bt_ref,
               r128_ref, r32_ref, r16_ref, sz_ref, st_ref, se_ref,
               wm_ref, b1_ref, w2_ref, b2_ref,
               out_ref, tf_scr, zm_scr):
    t = t_ref[:]
    for k in range(K):
        dtk = t - nt_ref[:, k:k + 1]
        tf_scr[:, k * TD:(k + 1) * TD] = jnp.cos(dtk * wt_ref[:] + bt_ref[:])
        gvk = g_ref[:, k:k + 1] >= 0
        zm_scr[:, k * D:(k + 1) * D] = jnp.where(
            gvk, zr_ref[:, k * D:(k + 1) * D], 0.0)
    tf = tf_scr[:]
    ef = ef_ref[:]
    zm = zm_scr[:]
    scores = (jnp.dot(zm, wsz_ref[:], preferred_element_type=f32)
              + jnp.dot(tf, wst_ref[:], preferred_element_type=f32)
              + jnp.dot(ef, wse_ref[:], preferred_element_type=f32))
    valid = m_ref[:] != 0
    ctxs = []
    for h in range(H):
        s_h = jnp.where(valid, scores[:, h * K:(h + 1) * K], -1e9)
        mx = jnp.max(s_h, axis=1, keepdims=True)
        p = jnp.exp(s_h - mx)
        a_h = p / jnp.sum(p, axis=1, keepdims=True)
        rep128 = jnp.dot(a_h, r128_ref[:], preferred_element_type=f32)
        rep32 = jnp.dot(a_h, r32_ref[:], preferred_element_type=f32)
        rep16 = jnp.dot(a_h, r16_ref[:], preferred_element_type=f32)
        cz = jnp.dot(zm * rep128, sz_ref[:], preferred_element_type=f32)
        ct = jnp.dot(tf * rep32, st_ref[:], preferred_element_type=f32)
        ce = jnp.dot(ef * rep16, se_ref[:], preferred_element_type=f32)
        ctxs += [cz, ce, ct]
    ctx2 = jnp.concatenate(ctxs, axis=1)
    y = jnp.maximum(jnp.dot(ctx2, wm_ref[:], preferred_element_type=f32)
                    + b1_ref[:], 0.0)
    out_ref[:] = jnp.dot(y, w2_ref[:], preferred_element_type=f32) + b2_ref[:]


def _hop0_call(t2d, nt, ef, msk, g2d, zrows, wsz, wst, wse, wt, bt,
               r128, r32, r16, sz, st, se, wm, b1, w2, b2):
    nblk = N0 // NB0
    full = lambda i: (0, 0)
    blk = lambda i: (i, 0)
    return pl.pallas_call(
        _hop0_body,
        grid=(nblk,),
        in_specs=[
            pl.BlockSpec((NB0, 1), blk), pl.BlockSpec((NB0, K), blk),
            pl.BlockSpec((NB0, K * ED), blk), pl.BlockSpec((NB0, K), blk),
            pl.BlockSpec((NB0, K), blk), pl.BlockSpec((NB0, K * D), blk),
            pl.BlockSpec((K * D, H * K), full),
            pl.BlockSpec((K * TD, H * K), full), pl.BlockSpec((K * ED, H * K), full),
            pl.BlockSpec((1, TD), full), pl.BlockSpec((1, TD), full),
            pl.BlockSpec((K, K * D), full),
            pl.BlockSpec((K, K * TD), full), pl.BlockSpec((K, K * ED), full),
            pl.BlockSpec((K * D, D), full),
            pl.BlockSpec((K * TD, TD), full), pl.BlockSpec((K * ED, ED), full),
            pl.BlockSpec((H * DKV, D), full), pl.BlockSpec((1, D), full),
            pl.BlockSpec((D, D), full), pl.BlockSpec((1, D), full),
        ],
        out_specs=pl.BlockSpec((NB0, D), blk),
        out_shape=jax.ShapeDtypeStruct((N0, D), f32),
        scratch_shapes=[pltpu.VMEM((NB0, K * TD), f32),
                        pltpu.VMEM((NB0, K * D), f32)],
    )(t2d, nt, ef, msk, g2d, zrows, wsz, wst, wse, wt, bt,
      r128, r32, r16, sz, st, se, wm, b1, w2, b2)


def _final_body(fs0_ref, fs1_ref, r0_ref, r1_ref,
                wsrc_ref, bsrc_ref, wdst_ref, bdst_ref, wout_ref, bout_ref,
                pos_ref, neg_ref):
    m0 = fs0_ref[:] >= 0
    m1 = fs1_ref[:] >= 0
    z = jnp.where(m0, r0_ref[:], jnp.where(m1, r1_ref[:], 0.0))
    zs = z[0:B]
    zd = z[B:2 * B]
    zn = z[2 * B:3 * B]
    a = jnp.dot(zs, wsrc_ref[:], preferred_element_type=f32) + bsrc_ref[:]
    hd = jnp.maximum(a + jnp.dot(zd, wdst_ref[:], preferred_element_type=f32)
                     + bdst_ref[:], 0.0)
    hn = jnp.maximum(a + jnp.dot(zn, wdst_ref[:], preferred_element_type=f32)
                     + bdst_ref[:], 0.0)
    lp = jnp.dot(hd, wout_ref[:], preferred_element_type=f32) + bout_ref[:]
    ln = jnp.dot(hn, wout_ref[:], preferred_element_type=f32) + bout_ref[:]
    pos_ref[:] = 1.0 / (1.0 + jnp.exp(-lp))
    neg_ref[:] = 1.0 / (1.0 + jnp.exp(-ln))


def _final_call(fs0, fs1, rows0, rows1, Wsrc, bsrc, Wdst, bdst, Wout, bout):
    return pl.pallas_call(
        _final_body,
        out_shape=[jax.ShapeDtypeStruct((B, 1), f32),
                   jax.ShapeDtypeStruct((B, 1), f32)],
    )(fs0, fs1, rows0, rows1, Wsrc, bsrc, Wdst, bdst, Wout, bout)


def _fold(Wq, Wk, Wv, W1, b_t):
    tvec = jnp.cos(b_t)
    qv = tvec @ Wq[D:, :]
    qh = qv.reshape(H, DH)
    wsc = jnp.einsum('khd,hd->kh', Wk.reshape(DKV, H, DH), qh)
    wsc = wsc / jnp.sqrt(jnp.asarray(DH, f32))
    Wvr = Wv.reshape(DKV, H, DH)
    return wsc, Wvr


def _score_mat(wsc_part, fdim):
    A = wsc_part[None, :, :, None] * jnp.eye(K, dtype=f32)[:, None, None, :]
    return A.reshape(K * fdim, H * K).astype(f32)


def kernel(nids0, nids1, times0, times1, nbr_times0, nbr_times1, nbr_feats0,
           nbr_feats1, nbr_mask0, nbr_mask1, nbr_nids_idx0, nid_to_idx,
           src_idx, dst_idx, neg_idx, w_t, b_t,
           Wq0, Wk0, Wv0, W1_0, b1_0, W2_0, b2_0,
           Wq1, Wk1, Wv1, W1_1, b1_1, W2_1, b2_1,
           Wsrc, bsrc, Wdst, bdst, Wout, bout):
    nids0 = nids0.astype(i32)
    nids1 = nids1.astype(i32)
    sel3 = jnp.concatenate([src_idx, dst_idx, neg_idx]).astype(i32)

    wsc1, Wv1r = _fold(Wq1, Wk1, Wv1, W1_1, b_t)
    wsc0, Wv0r = _fold(Wq0, Wk0, Wv0, W1_0, b_t)

    Wst1 = _score_mat(wsc1[D + ED:], TD)
    Wse1 = _score_mat(wsc1[D:D + ED], ED)
    Wsz0 = _score_mat(wsc0[:D], D)
    Wst0 = _score_mat(wsc0[D + ED:], TD)
    Wse0 = _score_mat(wsc0[D:D + ED], ED)

    eyeK = jnp.eye(K, dtype=f32)
    R32 = jnp.kron(eyeK, jnp.ones((1, TD), f32))
    R16 = jnp.kron(eyeK, jnp.ones((1, ED), f32))
    R128 = jnp.kron(eyeK, jnp.ones((1, D), f32))
    St = jnp.kron(jnp.ones((K, 1), f32), jnp.eye(TD, dtype=f32))
    Se = jnp.kron(jnp.ones((K, 1), f32), jnp.eye(ED, dtype=f32))
    Sz = jnp.kron(jnp.ones((K, 1), f32), jnp.eye(D, dtype=f32))

    sub1 = [jnp.concatenate([Wv1r[D:D + ED, h], Wv1r[D + ED:, h]], 0)
            for h in range(H)]
    Wvb1 = jnp.zeros((H * (ED + TD), D), f32)
    Wvb1 = Wvb1.at[0:48, 0:DH].set(sub1[0]).at[48:96, DH:D].set(sub1[1])
    Wm1 = Wvb1 @ W1_1[:D]

    Wvb0 = jnp.zeros((H * DKV, D), f32)
    Wvb0 = Wvb0.at[0:DKV, 0:DH].set(Wv0r[:, 0]).at[DKV:, DH:D].set(Wv0r[:, 1])
    Wm0 = Wvb0 @ W1_0[:D]

    wt2 = w_t.reshape(1, TD)
    bt2 = b_t.reshape(1, TD)

    out1 = _hop1_call(times1.reshape(N1, 1), nbr_times1,
                      nbr_feats1.reshape(N1, K * ED), nbr_mask1,
                      Wst1, Wse1, wt2, bt2, R32, R16, St, Se,
                      Wm1, b1_1.reshape(1, D), W2_1, b2_1.reshape(1, D))

    key1, key0 = _sc_keys()(nids1, nids0, nid_to_idx.astype(i32))
    winner1, winner0 = _sc_winner()(key1, key0)
    g0, nbr0rows, fs0, fs1 = _sc_compose()(
        winner1, winner0, nbr_nids_idx0.reshape(N1).astype(i32), sel3, out1)

    out0 = _hop0_call(times0.reshape(N0, 1), nbr_times0,
                      nbr_feats0.reshape(N0, K * ED), nbr_mask0,
                      g0.reshape(N0, K), nbr0rows.reshape(N0, K * D),
                      Wsz0, Wst0, Wse0, wt2, bt2, R128, R32, R16, Sz, St, Se,
                      Wm0, b1_0.reshape(1, D), W2_0, b2_0.reshape(1, D))

    rows0, rows1 = _sc_fgather()(fs0, fs1, out0, out1)
    pos, neg = _final_call(fs0.reshape(N0, 1), fs1.reshape(N0, 1),
                           rows0, rows1, Wsrc, bsrc.reshape(1, D),
                           Wdst, bdst.reshape(1, D), Wout, bout.reshape(1, 1))
    return (pos.reshape(B), neg.reshape(B))


# speedup vs baseline: 1.7979x; 1.7979x over previous
import functools

import jax
import jax.numpy as jnp
from jax import lax
from jax.experimental import pallas as pl
from jax.experimental.pallas import tpu as pltpu
from jax.experimental.pallas import tpu_sc as plsc

N0 = 3072; N1 = 30720; K = 10; U = 33792; NN = 100000; B = 1024
TD = 32; ED = 16; D = 128; H = 2; DH = D // H
DQ = D + TD; DKV = D + ED + TD

NW = 32
C1 = N1 // NW
C0 = N0 // NW
UW = U // NW

f32 = jnp.float32
i32 = jnp.int32

@functools.lru_cache(maxsize=1)
def _mesh():
    return plsc.VectorSubcoreMesh(core_axis_name="c", subcore_axis_name="s")


def _wid():
    return lax.axis_index("s") * 2 + lax.axis_index("c")


def _sc_keys_body(nids1_h, nids0_h, n2i_h, key1_h, key0_h,
                  nbuf, ibuf, n0buf, i0buf, sem):
    w = _wid()
    b1 = w * C1
    pltpu.sync_copy(nids1_h.at[pl.ds(b1, C1)], nbuf)
    pltpu.async_copy(n2i_h.at[nbuf], ibuf, sem).wait()
    pltpu.sync_copy(ibuf, key1_h.at[pl.ds(b1, C1)])

    b0 = w * C0
    pltpu.sync_copy(nids0_h.at[pl.ds(b0, C0)], n0buf)
    pltpu.async_copy(n2i_h.at[n0buf], i0buf, sem).wait()
    pltpu.sync_copy(i0buf, key0_h.at[pl.ds(b0, C0)])


@functools.lru_cache(maxsize=1)
def _sc_keys():
    return pl.kernel(
        _sc_keys_body,
        out_type=[jax.ShapeDtypeStruct((N1,), i32),
                  jax.ShapeDtypeStruct((N0,), i32)],
        mesh=_mesh(),
        compiler_params=pltpu.CompilerParams(needs_layout_passes=False),
        scratch_types=[pltpu.VMEM((C1,), i32), pltpu.VMEM((C1,), i32),
                       pltpu.VMEM((C0,), i32), pltpu.VMEM((C0,), i32),
                       pltpu.SemaphoreType.DMA],
    )


Z1 = N1 // 16
U16 = U // 16


def _sc_winner_body(key1_h, key0_h, w1_h, w0_h, kbuf, k0buf, wloc, acc, tmp,
                    shared):
    core = lax.axis_index("c")
    sid = lax.axis_index("s")
    iot = lax.iota(i32, 16)
    neg1 = jnp.full((16,), -1, i32)

    @pl.when(core == 0)
    def _winner1():
        pltpu.sync_copy(key1_h.at[pl.ds(sid * Z1, Z1)], kbuf)

        def init(c, _):
            wloc[pl.ds(c * 16, 16)] = neg1
            return 0

        lax.fori_loop(0, U // 16, init, 0)
        jb = sid * Z1

        def body(c, _):
            d = kbuf[pl.ds(c * 16, 16)]
            j = jb + c * 16 + iot
            plsc.store_scatter(wloc, [d], j)
            for _ in range(4):
                cur = plsc.load_gather(wloc, [d])
                plsc.store_scatter(wloc, [d], j, mask=cur < j)
            return 0

        lax.fori_loop(0, Z1 // 16, body, 0)
        pltpu.sync_copy(wloc, shared.at[pl.ds(sid * U, U)])

    @pl.when(core == 1)
    def _winner0():
        lo = sid * U16
        pltpu.sync_copy(key0_h, k0buf)

        def init(c, _):
            wloc[pl.ds(c * 16, 16)] = neg1
            return 0

        lax.fori_loop(0, U16 // 16, init, 0)

        def body(c, _):
            d = k0buf[pl.ds(c * 16, 16)]
            j = c * 16 + iot
            m = (d >= lo) & (d < lo + U16)
            dl = jnp.clip(d - lo, 0, U16 - 1)
            plsc.store_scatter(wloc, [dl], j, mask=m)
            for _ in range(4):
                cur = plsc.load_gather(wloc, [dl], mask=m)
                plsc.store_scatter(wloc, [dl], j, mask=m & (cur < j))
            return 0

        lax.fori_loop(0, N0 // 16, body, 0)
        pltpu.sync_copy(wloc.at[pl.ds(0, U16)], w0_h.at[pl.ds(lo, U16)])

    @pl.when(core == 0)
    def _combine():
        plsc.subcore_barrier()
        lo = sid * U16
        pltpu.sync_copy(shared.at[pl.ds(lo, U16)], acc)
        for t in range(1, 16):
            pltpu.sync_copy(shared.at[pl.ds(t * U + lo, U16)], tmp)

            def mx(c, _):
                acc[pl.ds(c * 16, 16)] = jnp.maximum(
                    acc[pl.ds(c * 16, 16)], tmp[pl.ds(c * 16, 16)])
                return 0

            lax.fori_loop(0, U16 // 16, mx, 0)
        pltpu.sync_copy(acc, w1_h.at[pl.ds(lo, U16)])


@functools.lru_cache(maxsize=1)
def _sc_winner():
    return pl.kernel(
        _sc_winner_body,
        out_type=[jax.ShapeDtypeStruct((U,), i32),
                  jax.ShapeDtypeStruct((U,), i32)],
        mesh=_mesh(),
        compiler_params=pltpu.CompilerParams(needs_layout_passes=False),
        scratch_types=[pltpu.VMEM((Z1,), i32), pltpu.VMEM((N0,), i32),
                       pltpu.VMEM((U,), i32), pltpu.VMEM((U16,), i32),
                       pltpu.VMEM((U16,), i32),
                       pltpu.VMEM_SHARED((16 * U,), i32)],
    )


_RG = 240


def _sc_compose_body(w1_h, w0_h, nbrf_h, sel3_h, out1_h,
                     g0_h, rows_h, fs0_h, fs1_h,
                     nfbuf, g0buf, gibuf, selbuf, f0buf, f1buf, rowbuf, sem):
    w = _wid()
    b1 = w * C1
    pltpu.sync_copy(nbrf_h.at[pl.ds(b1, C1)], nfbuf)
    pltpu.async_copy(w1_h.at[nfbuf], g0buf, sem).wait()
    pltpu.sync_copy(g0buf, g0_h.at[pl.ds(b1, C1)])

    iot = lax.iota(i32, 16)

    def clampb(c, _):
        g = g0buf[pl.ds(c * 16, 16)]
        gibuf[pl.ds(c * 16, 16)] = jnp.where(g < 0, b1 + c * 16 + iot, g)
        return 0

    lax.fori_loop(0, C1 // 16, clampb, 0)

    for s in range(C1 // _RG):
        idx = gibuf.at[pl.ds(s * _RG, _RG)]
        pltpu.async_copy(out1_h.at[idx], rowbuf, sem).wait()
        pltpu.sync_copy(rowbuf, rows_h.at[pl.ds(b1 + s * _RG, _RG)])

    b0 = w * C0
    pltpu.sync_copy(sel3_h.at[pl.ds(b0, C0)], selbuf)
    pltpu.async_copy(w0_h.at[selbuf], f0buf, sem).wait()
    pltpu.async_copy(w1_h.at[selbuf], f1buf, sem).wait()
    pltpu.sync_copy(f0buf, fs0_h.at[pl.ds(b0, C0)])
    pltpu.sync_copy(f1buf, fs1_h.at[pl.ds(b0, C0)])


@functools.lru_cache(maxsize=1)
def _sc_compose():
    return pl.kernel(
        _sc_compose_body,
        out_type=[jax.ShapeDtypeStruct((N1,), i32),
                  jax.ShapeDtypeStruct((N1, D), f32),
                  jax.ShapeDtypeStruct((N0,), i32),
                  jax.ShapeDtypeStruct((N0,), i32)],
        mesh=_mesh(),
        compiler_params=pltpu.CompilerParams(needs_layout_passes=False),
        scratch_types=[pltpu.VMEM((C1,), i32), pltpu.VMEM((C1,), i32),
                       pltpu.VMEM((C1,), i32), pltpu.VMEM((C0,), i32),
                       pltpu.VMEM((C0,), i32), pltpu.VMEM((C0,), i32),
                       pltpu.VMEM((_RG, D), f32),
                       pltpu.SemaphoreType.DMA],
    )


def _sc_fgather_body(fs0_h, fs1_h, out0_h, out1_h, r0_h, r1_h,
                     f0buf, f1buf, c0buf, c1buf, rb0, rb1, sem):
    w = _wid()
    b = w * C0
    pltpu.sync_copy(fs0_h.at[pl.ds(b, C0)], f0buf)
    pltpu.sync_copy(fs1_h.at[pl.ds(b, C0)], f1buf)

    iot = lax.iota(i32, 16)

    def clampb(c, _):
        spread = b + c * 16 + iot
        f0 = f0buf[pl.ds(c * 16, 16)]
        f1 = f1buf[pl.ds(c * 16, 16)]
        c0buf[pl.ds(c * 16, 16)] = jnp.where(f0 < 0, spread, f0)
        c1buf[pl.ds(c * 16, 16)] = jnp.where(f1 < 0, spread, f1)
        return 0

    lax.fori_loop(0, C0 // 16, clampb, 0)
    pltpu.async_copy(out0_h.at[c0buf], rb0, sem).wait()
    pltpu.sync_copy(rb0, r0_h.at[pl.ds(b, C0)])
    pltpu.async_copy(out1_h.at[c1buf], rb1, sem).wait()
    pltpu.sync_copy(rb1, r1_h.at[pl.ds(b, C0)])


@functools.lru_cache(maxsize=1)
def _sc_fgather():
    return pl.kernel(
        _sc_fgather_body,
        out_type=[jax.ShapeDtypeStruct((N0, D), f32),
                  jax.ShapeDtypeStruct((N0, D), f32)],
        mesh=_mesh(),
        compiler_params=pltpu.CompilerParams(needs_layout_passes=False),
        scratch_types=[pltpu.VMEM((C0,), i32), pltpu.VMEM((C0,), i32),
                       pltpu.VMEM((C0,), i32), pltpu.VMEM((C0,), i32),
                       pltpu.VMEM((C0, D), f32), pltpu.VMEM((C0, D), f32),
                       pltpu.SemaphoreType.DMA],
    )


NB1 = 1024


def _hop1_body(t_ref, nt_ref, ef_ref, m_ref,
               wst_ref, wse_ref, wt_ref, bt_ref,
               r32_ref, r16_ref, st_ref, se_ref,
               wm_ref, b1_ref, w2_ref, b2_ref,
               out_ref, tf_scr):
    t = t_ref[:]
    for k in range(K):
        dtk = t - nt_ref[:, k:k + 1]
        tf_scr[:, k * TD:(k + 1) * TD] = jnp.cos(dtk * wt_ref[:] + bt_ref[:])
    tf = tf_scr[:]
    ef = ef_ref[:]
    scores = (jnp.dot(tf, wst_ref[:], preferred_element_type=f32)
              + jnp.dot(ef, wse_ref[:], preferred_element_type=f32))
    valid = m_ref[:] != 0
    ctxs = []
    for h in range(H):
        s_h = jnp.where(valid, scores[:, h * K:(h + 1) * K], -1e9)
        mx = jnp.max(s_h, axis=1, keepdims=True)
        p = jnp.exp(s_h - mx)
        a_h = p / jnp.sum(p, axis=1, keepdims=True)
        rep32 = jnp.dot(a_h, r32_ref[:], preferred_element_type=f32)
        rep16 = jnp.dot(a_h, r16_ref[:], preferred_element_type=f32)
        ct = jnp.dot(tf * rep32, st_ref[:], preferred_element_type=f32)
        ce = jnp.dot(ef * rep16, se_ref[:], preferred_element_type=f32)
        ctxs += [ce, ct]
    ctx2 = jnp.concatenate(ctxs, axis=1)
    y = jnp.maximum(jnp.dot(ctx2, wm_ref[:], preferred_element_type=f32)
                    + b1_ref[:], 0.0)
    out_ref[:] = jnp.dot(y, w2_ref[:], preferred_element_type=f32) + b2_ref[:]


def _hop1_call(t2d, nt, ef, msk, wst, wse, wt, bt, r32, r16, st, se,
               wm, b1, w2, b2):
    nblk = N1 // NB1
    full = lambda i: (0, 0)
    blk = lambda i: (i, 0)
    return pl.pallas_call(
        _hop1_body,
        grid=(nblk,),
        in_specs=[
            pl.BlockSpec((NB1, 1), blk), pl.BlockSpec((NB1, K), blk),
            pl.BlockSpec((NB1, K * ED), blk), pl.BlockSpec((NB1, K), blk),
            pl.BlockSpec((K * TD, H * K), full), pl.BlockSpec((K * ED, H * K), full),
            pl.BlockSpec((1, TD), full), pl.BlockSpec((1, TD), full),
            pl.BlockSpec((K, K * TD), full), pl.BlockSpec((K, K * ED), full),
            pl.BlockSpec((K * TD, TD), full), pl.BlockSpec((K * ED, ED), full),
            pl.BlockSpec((2 * (ED + TD), D), full), pl.BlockSpec((1, D), full),
            pl.BlockSpec((D, D), full), pl.BlockSpec((1, D), full),
        ],
        out_specs=pl.BlockSpec((NB1, D), blk),
        out_shape=jax.ShapeDtypeStruct((N1, D), f32),
        scratch_shapes=[pltpu.VMEM((NB1, K * TD), f32)],
    )(t2d, nt, ef, msk, wst, wse, wt, bt, r32, r16, st, se, wm, b1, w2, b2)


NB0 = 512


def _hop0_body(t_ref, nt_ref, ef_ref, m_ref, g_ref, zr_ref,
               wsz_ref, wst_ref, wse_ref, wt_ref, bt_ref,
               r128_ref, r32_ref, r16_ref, sz_ref, st_ref, se_ref,
               wm_ref, b1_ref, w2_ref, b2_ref,
               out_ref, tf_scr, zm_scr):
    t = t_ref[:]
    for k in range(K):
        dtk = t - nt_ref[:, k:k + 1]
        tf_scr[:, k * TD:(k + 1) * TD] = jnp.cos(dtk * wt_ref[:] + bt_ref[:])
        gvk = g_ref[:, k:k + 1] >= 0
        zm_scr[:, k * D:(k + 1) * D] = jnp.where(
            gvk, zr_ref[:, k * D:(k + 1) * D], 0.0)
    tf = tf_scr[:]
    ef = ef_ref[:]
    zm = zm_scr[:]
    scores = (jnp.dot(zm, wsz_ref[:], preferred_element_type=f32)
              + jnp.dot(tf, wst_ref[:], preferred_element_type=f32)
              + jnp.dot(ef, wse_ref[:], preferred_element_type=f32))
    valid = m_ref[:] != 0
    ctxs = []
    for h in range(H):
        s_h = jnp.where(valid, scores[:, h * K:(h + 1) * K], -1e9)
        mx = jnp.max(s_h, axis=1, keepdims=True)
        p = jnp.exp(s_h - mx)
        a_h = p / jnp.sum(p, axis=1, keepdims=True)
        rep128 = jnp.dot(a_h, r128_ref[:], preferred_element_type=f32)
        rep32 = jnp.dot(a_h, r32_ref[:], preferred_element_type=f32)
        rep16 = jnp.dot(a_h, r16_ref[:], preferred_element_type=f32)
        cz = jnp.dot(zm * rep128, sz_ref[:], preferred_element_type=f32)
        ct = jnp.dot(tf * rep32, st_ref[:], preferred_element_type=f32)
        ce = jnp.dot(ef * rep16, se_ref[:], preferred_element_type=f32)
        ctxs += [cz, ce, ct]
    ctx2 = jnp.concatenate(ctxs, axis=1)
    y = jnp.maximum(jnp.dot(ctx2, wm_ref[:], preferred_element_type=f32)
                    + b1_ref[:], 0.0)
    out_ref[:] = jnp.dot(y, w2_ref[:], preferred_element_type=f32) + b2_ref[:]


def _hop0_call(t2d, nt, ef, msk, g2d, zrows, wsz, wst, wse, wt, bt,
               r128, r32, r16, sz, st, se, wm, b1, w2, b2):
    nblk = N0 // NB0
    full = lambda i: (0, 0)
    blk = lambda i: (i, 0)
    return pl.pallas_call(
        _hop0_body,
        grid=(nblk,),
        in_specs=[
            pl.BlockSpec((NB0, 1), blk), pl.BlockSpec((NB0, K), blk),
            pl.BlockSpec((NB0, K * ED), blk), pl.BlockSpec((NB0, K), blk),
            pl.BlockSpec((NB0, K), blk), pl.BlockSpec((NB0, K * D), blk),
            pl.BlockSpec((K * D, H * K), full),
            pl.BlockSpec((K * TD, H * K), full), pl.BlockSpec((K * ED, H * K), full),
            pl.BlockSpec((1, TD), full), pl.BlockSpec((1, TD), full),
            pl.BlockSpec((K, K * D), full),
            pl.BlockSpec((K, K * TD), full), pl.BlockSpec((K, K * ED), full),
            pl.BlockSpec((K * D, D), full),
            pl.BlockSpec((K * TD, TD), full), pl.BlockSpec((K * ED, ED), full),
            pl.BlockSpec((H * DKV, D), full), pl.BlockSpec((1, D), full),
            pl.BlockSpec((D, D), full), pl.BlockSpec((1, D), full),
        ],
        out_specs=pl.BlockSpec((NB0, D), blk),
        out_shape=jax.ShapeDtypeStruct((N0, D), f32),
        scratch_shapes=[pltpu.VMEM((NB0, K * TD), f32),
                        pltpu.VMEM((NB0, K * D), f32)],
    )(t2d, nt, ef, msk, g2d, zrows, wsz, wst, wse, wt, bt,
      r128, r32, r16, sz, st, se, wm, b1, w2, b2)


def _final_body(fs0_ref, fs1_ref, r0_ref, r1_ref,
                wsrc_ref, bsrc_ref, wdst_ref, bdst_ref, wout_ref, bout_ref,
                pos_ref, neg_ref):
    m0 = fs0_ref[:] >= 0
    m1 = fs1_ref[:] >= 0
    z = jnp.where(m0, r0_ref[:], jnp.where(m1, r1_ref[:], 0.0))
    zs = z[0:B]
    zd = z[B:2 * B]
    zn = z[2 * B:3 * B]
    a = jnp.dot(zs, wsrc_ref[:], preferred_element_type=f32) + bsrc_ref[:]
    hd = jnp.maximum(a + jnp.dot(zd, wdst_ref[:], preferred_element_type=f32)
                     + bdst_ref[:], 0.0)
    hn = jnp.maximum(a + jnp.dot(zn, wdst_ref[:], preferred_element_type=f32)
                     + bdst_ref[:], 0.0)
    lp = jnp.dot(hd, wout_ref[:], preferred_element_type=f32) + bout_ref[:]
    ln = jnp.dot(hn, wout_ref[:], preferred_element_type=f32) + bout_ref[:]
    pos_ref[:] = 1.0 / (1.0 + jnp.exp(-lp))
    neg_ref[:] = 1.0 / (1.0 + jnp.exp(-ln))


def _final_call(fs0, fs1, rows0, rows1, Wsrc, bsrc, Wdst, bdst, Wout, bout):
    return pl.pallas_call(
        _final_body,
        out_shape=[jax.ShapeDtypeStruct((B, 1), f32),
                   jax.ShapeDtypeStruct((B, 1), f32)],
    )(fs0, fs1, rows0, rows1, Wsrc, bsrc, Wdst, bdst, Wout, bout)


def _fold(Wq, Wk, Wv, W1, b_t):
    tvec = jnp.cos(b_t)
    qv = tvec @ Wq[D:, :]
    qh = qv.reshape(H, DH)
    wsc = jnp.einsum('khd,hd->kh', Wk.reshape(DKV, H, DH), qh)
    wsc = wsc / jnp.sqrt(jnp.asarray(DH, f32))
    Wvr = Wv.reshape(DKV, H, DH)
    return wsc, Wvr


def _score_mat(wsc_part, fdim):
    A = wsc_part[None, :, :, None] * jnp.eye(K, dtype=f32)[:, None, None, :]
    return A.reshape(K * fdim, H * K).astype(f32)


def kernel(nids0, nids1, times0, times1, nbr_times0, nbr_times1, nbr_feats0,
           nbr_feats1, nbr_mask0, nbr_mask1, nbr_nids_idx0, nid_to_idx,
           src_idx, dst_idx, neg_idx, w_t, b_t,
           Wq0, Wk0, Wv0, W1_0, b1_0, W2_0, b2_0,
           Wq1, Wk1, Wv1, W1_1, b1_1, W2_1, b2_1,
           Wsrc, bsrc, Wdst, bdst, Wout, bout):
    nids0 = nids0.astype(i32)
    nids1 = nids1.astype(i32)
    sel3 = jnp.concatenate([src_idx, dst_idx, neg_idx]).astype(i32)

    wsc1, Wv1r = _fold(Wq1, Wk1, Wv1, W1_1, b_t)
    wsc0, Wv0r = _fold(Wq0, Wk0, Wv0, W1_0, b_t)

    Wst1 = _score_mat(wsc1[D + ED:], TD)
    Wse1 = _score_mat(wsc1[D:D + ED], ED)
    Wsz0 = _score_mat(wsc0[:D], D)
    Wst0 = _score_mat(wsc0[D + ED:], TD)
    Wse0 = _score_mat(wsc0[D:D + ED], ED)

    eyeK = jnp.eye(K, dtype=f32)
    R32 = jnp.kron(eyeK, jnp.ones((1, TD), f32))
    R16 = jnp.kron(eyeK, jnp.ones((1, ED), f32))
    R128 = jnp.kron(eyeK, jnp.ones((1, D), f32))
    St = jnp.kron(jnp.ones((K, 1), f32), jnp.eye(TD, dtype=f32))
    Se = jnp.kron(jnp.ones((K, 1), f32), jnp.eye(ED, dtype=f32))
    Sz = jnp.kron(jnp.ones((K, 1), f32), jnp.eye(D, dtype=f32))

    sub1 = [jnp.concatenate([Wv1r[D:D + ED, h], Wv1r[D + ED:, h]], 0)
            for h in range(H)]
    Wvb1 = jnp.zeros((H * (ED + TD), D), f32)
    Wvb1 = Wvb1.at[0:48, 0:DH].set(sub1[0]).at[48:96, DH:D].set(sub1[1])
    Wm1 = Wvb1 @ W1_1[:D]

    Wvb0 = jnp.zeros((H * DKV, D), f32)
    Wvb0 = Wvb0.at[0:DKV, 0:DH].set(Wv0r[:, 0]).at[DKV:, DH:D].set(Wv0r[:, 1])
    Wm0 = Wvb0 @ W1_0[:D]

    wt2 = w_t.reshape(1, TD)
    bt2 = b_t.reshape(1, TD)

    out1 = _hop1_call(times1.reshape(N1, 1), nbr_times1,
                      nbr_feats1.reshape(N1, K * ED), nbr_mask1,
                      Wst1, Wse1, wt2, bt2, R32, R16, St, Se,
                      Wm1, b1_1.reshape(1, D), W2_1, b2_1.reshape(1, D))

    key1, key0 = _sc_keys()(nids1, nids0, nid_to_idx.astype(i32))
    winner1, winner0 = _sc_winner()(key1, key0)
    g0, nbr0rows, fs0, fs1 = _sc_compose()(
        winner1, winner0, nbr_nids_idx0.reshape(N1).astype(i32), sel3, out1)

    out0 = _hop0_call(times0.reshape(N0, 1), nbr_times0,
                      nbr_feats0.reshape(N0, K * ED), nbr_mask0,
                      g0.reshape(N0, K), nbr0rows.reshape(N0, K * D),
                      Wsz0, Wst0, Wse0, wt2, bt2, R128, R32, R16, Sz, St, Se,
                      Wm0, b1_0.reshape(1, D), W2_0, b2_0.reshape(1, D))

    rows0, rows1 = _sc_fgather()(fs0, fs1, out0, out1)
    pos, neg = _final_call(fs0.reshape(N0, 1), fs1.reshape(N0, 1),
                           rows0, rows1, Wsrc, bsrc.reshape(1, D),
                           Wdst, bdst.reshape(1, D), Wout, bout.reshape(1, 1))
    return (pos.reshape(B), neg.reshape(B))


# speedup vs baseline: 4.6810x; 2.6036x over previous
import functools

import jax
import jax.numpy as jnp
from jax import lax
from jax.experimental import pallas as pl
from jax.experimental.pallas import tpu as pltpu
from jax.experimental.pallas import tpu_sc as plsc

N0 = 3072; N1 = 30720; K = 10; U = 33792; NN = 100000; B = 1024
TD = 32; ED = 16; D = 128; H = 2; DH = D // H
DQ = D + TD; DKV = D + ED + TD

NW = 32
C1 = N1 // NW
C0 = N0 // NW
UW = U // NW

f32 = jnp.float32
i32 = jnp.int32

@functools.lru_cache(maxsize=1)
def _mesh():
    return plsc.VectorSubcoreMesh(core_axis_name="c", subcore_axis_name="s")


def _wid():
    return lax.axis_index("s") * 2 + lax.axis_index("c")


def _sc_keys_body(nids1_h, nids0_h, n2i_h, key1_h, key0_h,
                  nbuf, ibuf, n0buf, i0buf, sem):
    w = _wid()
    b1 = w * C1
    pltpu.sync_copy(nids1_h.at[pl.ds(b1, C1)], nbuf)
    pltpu.async_copy(n2i_h.at[nbuf], ibuf, sem).wait()
    pltpu.sync_copy(ibuf, key1_h.at[pl.ds(b1, C1)])

    b0 = w * C0
    pltpu.sync_copy(nids0_h.at[pl.ds(b0, C0)], n0buf)
    pltpu.async_copy(n2i_h.at[n0buf], i0buf, sem).wait()
    pltpu.sync_copy(i0buf, key0_h.at[pl.ds(b0, C0)])


@functools.lru_cache(maxsize=1)
def _sc_keys():
    return pl.kernel(
        _sc_keys_body,
        out_type=[jax.ShapeDtypeStruct((N1,), i32),
                  jax.ShapeDtypeStruct((N0,), i32)],
        mesh=_mesh(),
        compiler_params=pltpu.CompilerParams(needs_layout_passes=False),
        scratch_types=[pltpu.VMEM((C1,), i32), pltpu.VMEM((C1,), i32),
                       pltpu.VMEM((C0,), i32), pltpu.VMEM((C0,), i32),
                       pltpu.SemaphoreType.DMA],
    )


Z1 = N1 // 16
U16 = U // 16


def _sc_winner_body(key1_h, key0_h, w1_h, w0_h, kbuf, k0buf, wloc, acc, tmp,
                    shared):
    core = lax.axis_index("c")
    sid = lax.axis_index("s")
    iot = lax.iota(i32, 16)
    neg1 = jnp.full((16,), -1, i32)

    @pl.when(core == 0)
    def _winner1():
        pltpu.sync_copy(key1_h.at[pl.ds(sid * Z1, Z1)], kbuf)

        def init(c, _):
            wloc[pl.ds(c * 16, 16)] = neg1
            return 0

        lax.fori_loop(0, U // 16, init, 0)
        jb = sid * Z1

        def body(c, _):
            d = kbuf[pl.ds(c * 16, 16)]
            j = jb + c * 16 + iot
            plsc.store_scatter(wloc, [d], j)
            for _ in range(4):
                cur = plsc.load_gather(wloc, [d])
                plsc.store_scatter(wloc, [d], j, mask=cur < j)
            return 0

        lax.fori_loop(0, Z1 // 16, body, 0)
        pltpu.sync_copy(wloc, shared.at[pl.ds(sid * U, U)])

    @pl.when(core == 1)
    def _winner0():
        lo = sid * U16
        pltpu.sync_copy(key0_h, k0buf)

        def init(c, _):
            wloc[pl.ds(c * 16, 16)] = neg1
            return 0

        lax.fori_loop(0, U16 // 16, init, 0)

        def body(c, _):
            d = k0buf[pl.ds(c * 16, 16)]
            j = c * 16 + iot
            m = (d >= lo) & (d < lo + U16)
            dl = jnp.clip(d - lo, 0, U16 - 1)
            plsc.store_scatter(wloc, [dl], j, mask=m)
            for _ in range(4):
                cur = plsc.load_gather(wloc, [dl], mask=m)
                plsc.store_scatter(wloc, [dl], j, mask=m & (cur < j))
            return 0

        lax.fori_loop(0, N0 // 16, body, 0)
        pltpu.sync_copy(wloc.at[pl.ds(0, U16)], w0_h.at[pl.ds(lo, U16)])

    @pl.when(core == 0)
    def _combine():
        plsc.subcore_barrier()
        lo = sid * U16
        pltpu.sync_copy(shared.at[pl.ds(lo, U16)], acc)
        for t in range(1, 16):
            pltpu.sync_copy(shared.at[pl.ds(t * U + lo, U16)], tmp)

            def mx(c, _):
                acc[pl.ds(c * 16, 16)] = jnp.maximum(
                    acc[pl.ds(c * 16, 16)], tmp[pl.ds(c * 16, 16)])
                return 0

            lax.fori_loop(0, U16 // 16, mx, 0)
        pltpu.sync_copy(acc, w1_h.at[pl.ds(lo, U16)])


@functools.lru_cache(maxsize=1)
def _sc_winner():
    return pl.kernel(
        _sc_winner_body,
        out_type=[jax.ShapeDtypeStruct((U,), i32),
                  jax.ShapeDtypeStruct((U,), i32)],
        mesh=_mesh(),
        compiler_params=pltpu.CompilerParams(needs_layout_passes=False),
        scratch_types=[pltpu.VMEM((Z1,), i32), pltpu.VMEM((N0,), i32),
                       pltpu.VMEM((U,), i32), pltpu.VMEM((U16,), i32),
                       pltpu.VMEM((U16,), i32),
                       pltpu.VMEM_SHARED((16 * U,), i32)],
    )


_RG = 240


def _sc_compose_body(w1_h, w0_h, nbrf_h, sel3_h, out1_h,
                     g0_h, rows_h, fs0_h, fs1_h,
                     nfbuf, g0buf, gibuf, selbuf, f0buf, f1buf, rowbuf, sem):
    w = _wid()
    b1 = w * C1
    pltpu.sync_copy(nbrf_h.at[pl.ds(b1, C1)], nfbuf)
    pltpu.async_copy(w1_h.at[nfbuf], g0buf, sem).wait()
    pltpu.sync_copy(g0buf, g0_h.at[pl.ds(b1, C1)])

    iot = lax.iota(i32, 16)

    def clampb(c, _):
        g = g0buf[pl.ds(c * 16, 16)]
        gibuf[pl.ds(c * 16, 16)] = jnp.where(g < 0, b1 + c * 16 + iot, g)
        return 0

    lax.fori_loop(0, C1 // 16, clampb, 0)

    for s in range(C1 // _RG):
        idx = gibuf.at[pl.ds(s * _RG, _RG)]
        pltpu.async_copy(out1_h.at[idx], rowbuf, sem).wait()
        pltpu.sync_copy(rowbuf, rows_h.at[pl.ds(b1 + s * _RG, _RG)])

    b0 = w * C0
    pltpu.sync_copy(sel3_h.at[pl.ds(b0, C0)], selbuf)
    pltpu.async_copy(w0_h.at[selbuf], f0buf, sem).wait()
    pltpu.async_copy(w1_h.at[selbuf], f1buf, sem).wait()
    pltpu.sync_copy(f0buf, fs0_h.at[pl.ds(b0, C0)])
    pltpu.sync_copy(f1buf, fs1_h.at[pl.ds(b0, C0)])


@functools.lru_cache(maxsize=1)
def _sc_compose():
    return pl.kernel(
        _sc_compose_body,
        out_type=[jax.ShapeDtypeStruct((N1,), i32),
                  jax.ShapeDtypeStruct((N1, D), f32),
                  jax.ShapeDtypeStruct((N0,), i32),
                  jax.ShapeDtypeStruct((N0,), i32)],
        mesh=_mesh(),
        compiler_params=pltpu.CompilerParams(needs_layout_passes=False),
        scratch_types=[pltpu.VMEM((C1,), i32), pltpu.VMEM((C1,), i32),
                       pltpu.VMEM((C1,), i32), pltpu.VMEM((C0,), i32),
                       pltpu.VMEM((C0,), i32), pltpu.VMEM((C0,), i32),
                       pltpu.VMEM((_RG, D), f32),
                       pltpu.SemaphoreType.DMA],
    )


def _sc_fgather_body(fs0_h, fs1_h, out0_h, out1_h, r0_h, r1_h,
                     f0buf, f1buf, c0buf, c1buf, rb0, rb1, sem):
    w = _wid()
    b = w * C0
    pltpu.sync_copy(fs0_h.at[pl.ds(b, C0)], f0buf)
    pltpu.sync_copy(fs1_h.at[pl.ds(b, C0)], f1buf)

    iot = lax.iota(i32, 16)

    def clampb(c, _):
        spread = b + c * 16 + iot
        f0 = f0buf[pl.ds(c * 16, 16)]
        f1 = f1buf[pl.ds(c * 16, 16)]
        c0buf[pl.ds(c * 16, 16)] = jnp.where(f0 < 0, spread, f0)
        c1buf[pl.ds(c * 16, 16)] = jnp.where(f1 < 0, spread, f1)
        return 0

    lax.fori_loop(0, C0 // 16, clampb, 0)
    pltpu.async_copy(out0_h.at[c0buf], rb0, sem).wait()
    pltpu.sync_copy(rb0, r0_h.at[pl.ds(b, C0)])
    pltpu.async_copy(out1_h.at[c1buf], rb1, sem).wait()
    pltpu.sync_copy(rb1, r1_h.at[pl.ds(b, C0)])


@functools.lru_cache(maxsize=1)
def _sc_fgather():
    return pl.kernel(
        _sc_fgather_body,
        out_type=[jax.ShapeDtypeStruct((N0, D), f32),
                  jax.ShapeDtypeStruct((N0, D), f32)],
        mesh=_mesh(),
        compiler_params=pltpu.CompilerParams(needs_layout_passes=False),
        scratch_types=[pltpu.VMEM((C0,), i32), pltpu.VMEM((C0,), i32),
                       pltpu.VMEM((C0,), i32), pltpu.VMEM((C0,), i32),
                       pltpu.VMEM((C0, D), f32), pltpu.VMEM((C0, D), f32),
                       pltpu.SemaphoreType.DMA],
    )


NB1 = 1024


def _cos_poly(z):
    c = (0.9999999999330123, -0.4999999993200308, 0.041666665529783105,
         -0.0013888881564023522, 2.4801350571971984e-05, -2.7553012142874234e-07,
         2.083054869783779e-09, -1.11796086279586e-11, 3.7715016220347143e-14)
    acc = jnp.full_like(z, c[8])
    for i in range(7, -1, -1):
        acc = acc * z + c[i]
    return acc


def _hop1_body(t_ref, nt_ref, ef_ref, m_ref,
               wst_ref, wse_ref, wt10_ref, bt10_ref,
               r32_ref, r16_ref, st_ref, se_ref,
               wm_ref, b1_ref, w2_ref, b2_ref,
               out_ref):
    t = t_ref[:]
    ntr = jnp.dot(nt_ref[:], r32_ref[:], preferred_element_type=f32)
    x = (t - ntr) * wt10_ref[:] + bt10_ref[:]
    tf = _cos_poly(x * x)
    ef = ef_ref[:]
    scores = (jnp.dot(tf, wst_ref[:], preferred_element_type=f32)
              + jnp.dot(ef, wse_ref[:], preferred_element_type=f32))
    valid = m_ref[:] != 0
    ctxs = []
    for h in range(H):
        s_h = jnp.where(valid, scores[:, h * K:(h + 1) * K], -1e9)
        mx = jnp.max(s_h, axis=1, keepdims=True)
        p = jnp.exp(s_h - mx)
        a_h = p / jnp.sum(p, axis=1, keepdims=True)
        rep32 = jnp.dot(a_h, r32_ref[:], preferred_element_type=f32)
        rep16 = jnp.dot(a_h, r16_ref[:], preferred_element_type=f32)
        ct = jnp.dot(tf * rep32, st_ref[:], preferred_element_type=f32)
        ce = jnp.dot(ef * rep16, se_ref[:], preferred_element_type=f32)
        ctxs += [ce, ct]
    ctx2 = jnp.concatenate(ctxs, axis=1)
    y = jnp.maximum(jnp.dot(ctx2, wm_ref[:], preferred_element_type=f32)
                    + b1_ref[:], 0.0)
    out_ref[:] = jnp.dot(y, w2_ref[:], preferred_element_type=f32) + b2_ref[:]


def _hop1_call(t2d, nt, ef, msk, wst, wse, wt10, bt10, r32, r16, st, se,
               wm, b1, w2, b2):
    nblk = N1 // NB1
    full = lambda i: (0, 0)
    blk = lambda i: (i, 0)
    return pl.pallas_call(
        _hop1_body,
        grid=(nblk,),
        in_specs=[
            pl.BlockSpec((NB1, 1), blk), pl.BlockSpec((NB1, K), blk),
            pl.BlockSpec((NB1, K * ED), blk), pl.BlockSpec((NB1, K), blk),
            pl.BlockSpec((K * TD, H * K), full), pl.BlockSpec((K * ED, H * K), full),
            pl.BlockSpec((1, K * TD), full), pl.BlockSpec((1, K * TD), full),
            pl.BlockSpec((K, K * TD), full), pl.BlockSpec((K, K * ED), full),
            pl.BlockSpec((K * TD, TD), full), pl.BlockSpec((K * ED, ED), full),
            pl.BlockSpec((2 * (ED + TD), D), full), pl.BlockSpec((1, D), full),
            pl.BlockSpec((D, D), full), pl.BlockSpec((1, D), full),
        ],
        out_specs=pl.BlockSpec((NB1, D), blk),
        out_shape=jax.ShapeDtypeStruct((N1, D), f32),
    )(t2d, nt, ef, msk, wst, wse, wt10, bt10, r32, r16, st, se, wm, b1, w2, b2)


NB0 = 512


def _hop0_body(t_ref, nt_ref, ef_ref, m_ref, gf_ref, zr_ref,
               wszv_ref, wst_ref, wse_ref, wt10_ref, bt10_ref,
               r32_ref, r16_ref, st_ref, se_ref,
               wm_ref, b1_ref, w2_ref, b2_ref,
               out_ref):
    t = t_ref[:]
    ntr = jnp.dot(nt_ref[:], r32_ref[:], preferred_element_type=f32)
    x = (t - ntr) * wt10_ref[:] + bt10_ref[:]
    tf = _cos_poly(x * x)
    ef = ef_ref[:]
    zm = jnp.where(gf_ref[:] >= 0, zr_ref[:], 0.0)
    zm3 = zm.reshape(NB0, K, D)
    scores = (jnp.dot(tf, wst_ref[:], preferred_element_type=f32)
              + jnp.dot(ef, wse_ref[:], preferred_element_type=f32))
    valid = m_ref[:] != 0
    ctxs = []
    for h in range(H):
        sz = jnp.sum(zm3 * wszv_ref[h][None, None, :], axis=2)
        s_h = jnp.where(valid, scores[:, h * K:(h + 1) * K] + sz, -1e9)
        mx = jnp.max(s_h, axis=1, keepdims=True)
        p = jnp.exp(s_h - mx)
        a_h = p / jnp.sum(p, axis=1, keepdims=True)
        cz = jnp.sum(zm3 * a_h[:, :, None], axis=1)
        rep32 = jnp.dot(a_h, r32_ref[:], preferred_element_type=f32)
        rep16 = jnp.dot(a_h, r16_ref[:], preferred_element_type=f32)
        ct = jnp.dot(tf * rep32, st_ref[:], preferred_element_type=f32)
        ce = jnp.dot(ef * rep16, se_ref[:], preferred_element_type=f32)
        ctxs += [cz, ce, ct]
    ctx2 = jnp.concatenate(ctxs, axis=1)
    y = jnp.maximum(jnp.dot(ctx2, wm_ref[:], preferred_element_type=f32)
                    + b1_ref[:], 0.0)
    out_ref[:] = jnp.dot(y, w2_ref[:], preferred_element_type=f32) + b2_ref[:]


def _hop0_call(t2d, nt, ef, msk, gf, zrows, wszv, wst, wse, wt10, bt10,
               r32, r16, st, se, wm, b1, w2, b2):
    nblk = N0 // NB0
    full = lambda i: (0, 0)
    blk = lambda i: (i, 0)
    return pl.pallas_call(
        _hop0_body,
        grid=(nblk,),
        in_specs=[
            pl.BlockSpec((NB0, 1), blk), pl.BlockSpec((NB0, K), blk),
            pl.BlockSpec((NB0, K * ED), blk), pl.BlockSpec((NB0, K), blk),
            pl.BlockSpec((NB0 * K, 1), blk), pl.BlockSpec((NB0 * K, D), blk),
            pl.BlockSpec((H, D), full),
            pl.BlockSpec((K * TD, H * K), full), pl.BlockSpec((K * ED, H * K), full),
            pl.BlockSpec((1, K * TD), full), pl.BlockSpec((1, K * TD), full),
            pl.BlockSpec((K, K * TD), full), pl.BlockSpec((K, K * ED), full),
            pl.BlockSpec((K * TD, TD), full), pl.BlockSpec((K * ED, ED), full),
            pl.BlockSpec((H * DKV, D), full), pl.BlockSpec((1, D), full),
            pl.BlockSpec((D, D), full), pl.BlockSpec((1, D), full),
        ],
        out_specs=pl.BlockSpec((NB0, D), blk),
        out_shape=jax.ShapeDtypeStruct((N0, D), f32),
    )(t2d, nt, ef, msk, gf, zrows, wszv, wst, wse, wt10, bt10,
      r32, r16, st, se, wm, b1, w2, b2)


def _final_body(fs0_ref, fs1_ref, r0_ref, r1_ref,
                wsrc_ref, bsrc_ref, wdst_ref, bdst_ref, wout_ref, bout_ref,
                pos_ref, neg_ref):
    m0 = fs0_ref[:] >= 0
    m1 = fs1_ref[:] >= 0
    z = jnp.where(m0, r0_ref[:], jnp.where(m1, r1_ref[:], 0.0))
    zs = z[0:B]
    zd = z[B:2 * B]
    zn = z[2 * B:3 * B]
    a = jnp.dot(zs, wsrc_ref[:], preferred_element_type=f32) + bsrc_ref[:]
    hd = jnp.maximum(a + jnp.dot(zd, wdst_ref[:], preferred_element_type=f32)
                     + bdst_ref[:], 0.0)
    hn = jnp.maximum(a + jnp.dot(zn, wdst_ref[:], preferred_element_type=f32)
                     + bdst_ref[:], 0.0)
    lp = jnp.dot(hd, wout_ref[:], preferred_element_type=f32) + bout_ref[:]
    ln = jnp.dot(hn, wout_ref[:], preferred_element_type=f32) + bout_ref[:]
    pos_ref[:] = 1.0 / (1.0 + jnp.exp(-lp))
    neg_ref[:] = 1.0 / (1.0 + jnp.exp(-ln))


def _final_call(fs0, fs1, rows0, rows1, Wsrc, bsrc, Wdst, bdst, Wout, bout):
    return pl.pallas_call(
        _final_body,
        out_shape=[jax.ShapeDtypeStruct((B, 1), f32),
                   jax.ShapeDtypeStruct((B, 1), f32)],
    )(fs0, fs1, rows0, rows1, Wsrc, bsrc, Wdst, bdst, Wout, bout)


def _fold(Wq, Wk, Wv, W1, b_t):
    tvec = jnp.cos(b_t)
    qv = tvec @ Wq[D:, :]
    qh = qv.reshape(H, DH)
    wsc = jnp.einsum('khd,hd->kh', Wk.reshape(DKV, H, DH), qh)
    wsc = wsc / jnp.sqrt(jnp.asarray(DH, f32))
    Wvr = Wv.reshape(DKV, H, DH)
    return wsc, Wvr


def _score_mat(wsc_part, fdim):
    A = wsc_part[None, :, :, None] * jnp.eye(K, dtype=f32)[:, None, None, :]
    return A.reshape(K * fdim, H * K).astype(f32)


def kernel(nids0, nids1, times0, times1, nbr_times0, nbr_times1, nbr_feats0,
           nbr_feats1, nbr_mask0, nbr_mask1, nbr_nids_idx0, nid_to_idx,
           src_idx, dst_idx, neg_idx, w_t, b_t,
           Wq0, Wk0, Wv0, W1_0, b1_0, W2_0, b2_0,
           Wq1, Wk1, Wv1, W1_1, b1_1, W2_1, b2_1,
           Wsrc, bsrc, Wdst, bdst, Wout, bout):
    nids0 = nids0.astype(i32)
    nids1 = nids1.astype(i32)
    sel3 = jnp.concatenate([src_idx, dst_idx, neg_idx]).astype(i32)

    wsc1, Wv1r = _fold(Wq1, Wk1, Wv1, W1_1, b_t)
    wsc0, Wv0r = _fold(Wq0, Wk0, Wv0, W1_0, b_t)

    Wst1 = _score_mat(wsc1[D + ED:], TD)
    Wse1 = _score_mat(wsc1[D:D + ED], ED)
    wszv0 = wsc0[:D].T
    Wst0 = _score_mat(wsc0[D + ED:], TD)
    Wse0 = _score_mat(wsc0[D:D + ED], ED)

    eyeK = jnp.eye(K, dtype=f32)
    R32 = jnp.kron(eyeK, jnp.ones((1, TD), f32))
    R16 = jnp.kron(eyeK, jnp.ones((1, ED), f32))
    St = jnp.kron(jnp.ones((K, 1), f32), jnp.eye(TD, dtype=f32))
    Se = jnp.kron(jnp.ones((K, 1), f32), jnp.eye(ED, dtype=f32))

    sub1 = [jnp.concatenate([Wv1r[D:D + ED, h], Wv1r[D + ED:, h]], 0)
            for h in range(H)]
    Wvb1 = jnp.zeros((H * (ED + TD), D), f32)
    Wvb1 = Wvb1.at[0:48, 0:DH].set(sub1[0]).at[48:96, DH:D].set(sub1[1])
    Wm1 = Wvb1 @ W1_1[:D]

    Wvb0 = jnp.zeros((H * DKV, D), f32)
    Wvb0 = Wvb0.at[0:DKV, 0:DH].set(Wv0r[:, 0]).at[DKV:, DH:D].set(Wv0r[:, 1])
    Wm0 = Wvb0 @ W1_0[:D]

    wt10 = jnp.tile(w_t, K).reshape(1, K * TD)
    bt10 = jnp.tile(b_t, K).reshape(1, K * TD)

    out1 = _hop1_call(times1.reshape(N1, 1), nbr_times1,
                      nbr_feats1.reshape(N1, K * ED), nbr_mask1,
                      Wst1, Wse1, wt10, bt10, R32, R16, St, Se,
                      Wm1, b1_1.reshape(1, D), W2_1, b2_1.reshape(1, D))

    key1, key0 = _sc_keys()(nids1, nids0, nid_to_idx.astype(i32))
    winner1, winner0 = _sc_winner()(key1, key0)
    g0, nbr0rows, fs0, fs1 = _sc_compose()(
        winner1, winner0, nbr_nids_idx0.reshape(N1).astype(i32), sel3, out1)

    out0 = _hop0_call(times0.reshape(N0, 1), nbr_times0,
                      nbr_feats0.reshape(N0, K * ED), nbr_mask0,
                      g0.reshape(N1, 1), nbr0rows,
                      wszv0, Wst0, Wse0, wt10, bt10, R32, R16, St, Se,
                      Wm0, b1_0.reshape(1, D), W2_0, b2_0.reshape(1, D))

    rows0, rows1 = _sc_fgather()(fs0, fs1, out0, out1)
    pos, neg = _final_call(fs0.reshape(N0, 1), fs1.reshape(N0, 1),
                           rows0, rows1, Wsrc, bsrc.reshape(1, D),
                           Wdst, bdst.reshape(1, D), Wout, bout.reshape(1, 1))
    return (pos.reshape(B), neg.reshape(B))
